# baseline (speedup 1.0000x reference)
import functools

import jax
import jax.numpy as jnp
from jax import lax
from jax.experimental import pallas as pl
from jax.experimental.pallas import tpu as pltpu
from jax.experimental.pallas import tpu_sc as plsc

NS = 64
NV = 16
HID = 192
N_NODES = 10000
N_EDGES = 160000

WEXT = NS + 3 * NV
MROW = 128
NTILES = 32
CHUNK = 64
EDGES_PER_TILE = N_EDGES // NTILES
FULL_ITERS = EDGES_PER_TILE // CHUNK
PAIRS = FULL_ITERS // 2
TAIL = EDGES_PER_TILE - FULL_ITERS * CHUNK
ACC_ROWS = 10112
ROWS_PER_TILE = ACC_ROWS // 16
MLP_BLK = 1280
XW = 128


def _mlp_body(at_ref, sht_ref, w1_ref, b1_ref, w2_ref, b2_ref, o_ref):
    at = at_ref[...].astype(jnp.bfloat16)
    w1 = w1_ref[...].astype(jnp.bfloat16)
    h = jnp.maximum(
        lax.dot_general(at, w1, (((0,), (0,)), ((), ())),
                        preferred_element_type=jnp.float32) + b1_ref[...], 0.0)
    w = jnp.dot(h.astype(jnp.bfloat16), w2_ref[...].astype(jnp.bfloat16),
                preferred_element_type=jnp.float32) + b2_ref[...]
    ws = w[:, :NS]
    wv = w[:, NS:NS + NV]
    er = lax.broadcasted_iota(jnp.int32, (4, 3), 0)
    ec = lax.broadcasted_iota(jnp.int32, (4, 3), 1)
    sel = (er == ec + 1).astype(jnp.float32)
    sh3 = lax.dot_general(sht_ref[...], sel, (((0,), (0,)), ((), ())),
                          preferred_element_type=jnp.float32)
    u0 = wv * sh3[:, 0:1]
    u1 = wv * sh3[:, 1:2]
    u2 = wv * sh3[:, 2:3]
    pad = jnp.zeros((MLP_BLK, MROW - WEXT), jnp.float32)
    o_ref[...] = jnp.concatenate([ws, u0, u1, u2, pad], axis=1)


def _mlp(edge_attr_t, edge_sh_t, W1, b1, W2, b2):
    grid = (N_EDGES // MLP_BLK,)
    return pl.pallas_call(
        _mlp_body,
        grid=grid,
        in_specs=[
            pl.BlockSpec((HID, MLP_BLK), lambda i: (0, i)),
            pl.BlockSpec((4, MLP_BLK), lambda i: (0, i)),
            pl.BlockSpec((HID, HID), lambda i: (0, 0)),
            pl.BlockSpec((1, HID), lambda i: (0, 0)),
            pl.BlockSpec((HID, NS + NV), lambda i: (0, 0)),
            pl.BlockSpec((1, NS + NV), lambda i: (0, 0)),
        ],
        out_specs=pl.BlockSpec((MLP_BLK, MROW), lambda i: (i, 0)),
        out_shape=jax.ShapeDtypeStruct((N_EDGES, MROW), jnp.float32),
    )(edge_attr_t, edge_sh_t, W1, b1, W2, b2)


def _sc_kernel(x, src, dst, wext):
    mesh = plsc.VectorSubcoreMesh(core_axis_name="c", subcore_axis_name="s")

    @functools.partial(
        pl.kernel,
        mesh=mesh,
        out_type=jax.ShapeDtypeStruct((2, ACC_ROWS, MROW), jnp.float32),
        scratch_types=[
            pltpu.VMEM((CHUNK,), jnp.int32),
            pltpu.VMEM((CHUNK,), jnp.int32),
            pltpu.VMEM((CHUNK,), jnp.int32),
            pltpu.VMEM((CHUNK,), jnp.int32),
            pltpu.VMEM((TAIL,), jnp.int32),
            pltpu.VMEM((TAIL,), jnp.int32),
            pltpu.VMEM((CHUNK, XW), jnp.float32),
            pltpu.VMEM((CHUNK, XW), jnp.float32),
            pltpu.VMEM((CHUNK, MROW), jnp.float32),
            pltpu.VMEM((CHUNK, MROW), jnp.float32),
            pltpu.VMEM((CHUNK, MROW), jnp.float32),
            pltpu.VMEM((CHUNK, MROW), jnp.float32),
            pltpu.VMEM_SHARED((ACC_ROWS, MROW), jnp.float32),
            pltpu.SemaphoreType.DMA,
            pltpu.SemaphoreType.DMA,
            pltpu.SemaphoreType.DMA,
            pltpu.SemaphoreType.DMA,
            pltpu.SemaphoreType.DMA,
            pltpu.SemaphoreType.DMA,
            pltpu.SemaphoreType.DMA,
            pltpu.SemaphoreType.DMA,
            pltpu.SemaphoreType.DMA,
        ],
    )
    def body(x_hbm, src_hbm, dst_hbm, w_hbm, out_hbm,
             idxs0, idxs1, idxd0, idxd1, idxs8, idxd8,
             hsrc0, hsrc1, w0, w1, msg0, msg1, acc_sh,
             sem_in0, sem_in1, sem_w0, sem_w1, sem_g0, sem_g1,
             sem_s0, sem_s1, sem):
        cid = lax.axis_index("c")
        sid = lax.axis_index("s")
        idxs = (idxs0, idxs1)
        idxd = (idxd0, idxd1)
        hsrc = (hsrc0, hsrc1)
        wv = (w0, w1)
        msg = (msg0, msg1)
        sem_in = (sem_in0, sem_in1)
        sem_w = (sem_w0, sem_w1)
        sem_g = (sem_g0, sem_g1)
        sem_s = (sem_s0, sem_s1)

        zeros16 = jnp.zeros((16,), jnp.float32)

        def zero_buf(buf):
            def f(i, _):
                r = i // (MROW // 16)
                c = i % (MROW // 16)
                buf[r, pl.ds(c * 16, 16)] = zeros16
                return 0
            lax.fori_loop(0, CHUNK * (MROW // 16), f, 0)

        zero_buf(msg0)
        zero_buf(msg1)

        for k in range(ROWS_PER_TILE // CHUNK):
            pltpu.sync_copy(
                msg0, acc_sh.at[pl.ds(sid * ROWS_PER_TILE + k * CHUNK, CHUNK)])
        rem = ROWS_PER_TILE % CHUNK
        if rem:
            pltpu.sync_copy(
                msg0.at[pl.ds(0, rem)],
                acc_sh.at[pl.ds(sid * ROWS_PER_TILE
                                + (ROWS_PER_TILE // CHUNK) * CHUNK, rem)])

        ii = lax.broadcasted_iota(jnp.int32, (16,), 0)

        cvec = jnp.where(ii == 0, 1.0, 0.0).astype(jnp.float32)

        def crow(buf):
            def f(r, _):
                buf[r, pl.ds(WEXT, 16)] = cvec
                return 0
            lax.fori_loop(0, CHUNK, f, 0)

        crow(msg0)
        crow(msg1)

        plsc.subcore_barrier()

        tbase = (cid * 16 + sid) * EDGES_PER_TILE

        def make_edge_body(hs, wb, mb):
            def edge_body(e, _):
                h0 = hs[e, pl.ds(0, 16)]
                mb[e, pl.ds(0, 16)] = h0 * wb[e, pl.ds(0, 16)]
                for k in range(1, NS // 16):
                    mb[e, pl.ds(k * 16, 16)] = (
                        hs[e, pl.ds(k * 16, 16)] * wb[e, pl.ds(k * 16, 16)])
                for j in range(3):
                    off = NS + j * 16
                    mb[e, pl.ds(off, 16)] = h0 * wb[e, pl.ds(off, 16)]
                return 0
            return edge_body

        def half_step(p, c, S, prefetch):
            pltpu.make_async_copy(x_hbm.at[idxs[S]], hsrc[S], sem_g[S]).wait()
            pltpu.make_async_copy(
                w_hbm.at[pl.ds(0, CHUNK)], wv[S], sem_w[S]).wait()

            @pl.when(p > 0)
            def _():
                pltpu.make_async_copy(
                    msg[S], acc_sh.at[idxd[S]], sem_s[S]).wait()

            base = tbase + c * CHUNK
            pltpu.sync_copy(dst_hbm.at[pl.ds(base, CHUNK)], idxd[S])

            @pl.when(prefetch)
            def _():
                nbase = tbase + (c + 2) * CHUNK
                pltpu.async_copy(
                    src_hbm.at[pl.ds(nbase, CHUNK)], idxs[S], sem_in[S])

            lax.fori_loop(0, CHUNK, make_edge_body(hsrc[S], wv[S], msg[S]), 0)

            pltpu.async_copy(msg[S], acc_sh.at[idxd[S]], sem_s[S], add=True)

            @pl.when(prefetch)
            def _():
                nbase = tbase + (c + 2) * CHUNK
                pltpu.async_copy(
                    w_hbm.at[pl.ds(nbase, CHUNK)], wv[S], sem_w[S])
                pltpu.make_async_copy(
                    src_hbm.at[pl.ds(nbase, CHUNK)], idxs[S], sem_in[S]).wait()
                pltpu.async_copy(x_hbm.at[idxs[S]], hsrc[S], sem_g[S])

        def pair_body(p, _):
            half_step(p, 2 * p, 0, p < PAIRS - 1)
            half_step(p, 2 * p + 1, 1, p < PAIRS - 1)
            return 0

        for S in range(2):
            base = tbase + S * CHUNK
            pltpu.async_copy(src_hbm.at[pl.ds(base, CHUNK)], idxs[S], sem_in[S])
            pltpu.async_copy(w_hbm.at[pl.ds(base, CHUNK)], wv[S], sem_w[S])
        for S in range(2):
            pltpu.make_async_copy(
                src_hbm.at[pl.ds(tbase, CHUNK)], idxs[S], sem_in[S]).wait()
            pltpu.async_copy(x_hbm.at[idxs[S]], hsrc[S], sem_g[S])

        lax.fori_loop(0, PAIRS, pair_body, 0)

        pltpu.make_async_copy(msg[0], acc_sh.at[idxd[0]], sem_s[0]).wait()
        pltpu.make_async_copy(msg[1], acc_sh.at[idxd[1]], sem_s[1]).wait()

        tb = tbase + FULL_ITERS * CHUNK
        pltpu.sync_copy(src_hbm.at[pl.ds(tb, TAIL)], idxs8)
        pltpu.sync_copy(dst_hbm.at[pl.ds(tb, TAIL)], idxd8)
        pltpu.sync_copy(w_hbm.at[pl.ds(tb, TAIL)], w0.at[pl.ds(0, TAIL)])
        pltpu.async_copy(x_hbm.at[idxs8], hsrc0.at[pl.ds(0, TAIL)], sem).wait()
        lax.fori_loop(0, TAIL, make_edge_body(hsrc0, w0, msg0), 0)
        pltpu.sync_copy(msg0.at[pl.ds(0, TAIL)], acc_sh.at[idxd8], add=True)

        plsc.subcore_barrier()

        pltpu.sync_copy(
            acc_sh.at[pl.ds(sid * ROWS_PER_TILE, ROWS_PER_TILE)],
            out_hbm.at[cid, pl.ds(sid * ROWS_PER_TILE, ROWS_PER_TILE)])

    return body(x, src, dst, wext)


def _combine_body(p_ref, o_ref):
    a = p_ref[0] + p_ref[1]
    deg = jnp.maximum(a[:, WEXT:WEXT + 1], 1.0)
    s = a[:, :NS] / deg
    v = a[:, NS:WEXT] / deg
    r = lax.broadcasted_iota(jnp.int32, (3 * NV, 3 * NV), 0)
    c = lax.broadcasted_iota(jnp.int32, (3 * NV, 3 * NV), 1)
    perm = ((c % 3) * NV + (c // 3) == r).astype(jnp.float32)
    vp = jnp.dot(v, perm, preferred_element_type=jnp.float32)
    o_ref[...] = jnp.concatenate([s, vp], axis=1)


def _combine(partials):
    grid = (10,)
    blk = N_NODES // 10
    return pl.pallas_call(
        _combine_body,
        grid=grid,
        in_specs=[pl.BlockSpec((2, blk, MROW), lambda i: (0, i, 0))],
        out_specs=pl.BlockSpec((blk, NS + 3 * NV), lambda i: (i, 0)),
        out_shape=jax.ShapeDtypeStruct((N_NODES, NS + 3 * NV), jnp.float32),
    )(partials)


def kernel(x, edge_index, edge_attr, edge_sh, W1, b1, W2, b2):
    src = edge_index[0].astype(jnp.int32)
    dst = edge_index[1].astype(jnp.int32)
    wext = _mlp(edge_attr.T, edge_sh.T, W1, b1.reshape(1, HID),
                W2, b2.reshape(1, NS + NV))
    x_pad = jnp.pad(x, ((0, 0), (0, XW - NS)))
    partials = _sc_kernel(x_pad, src, dst, wext)
    return _combine(partials)


# speedup vs baseline: 1.1244x; 1.1244x over previous
import functools

import jax
import jax.numpy as jnp
from jax import lax
from jax.experimental import pallas as pl
from jax.experimental.pallas import tpu as pltpu
from jax.experimental.pallas import tpu_sc as plsc

NS = 64
NV = 16
HID = 192
N_NODES = 10000
N_EDGES = 160000

WEXT = NS + 3 * NV
MROW = 128
NTILES = 32
CHUNK = 64
EDGES_PER_TILE = N_EDGES // NTILES
FULL_ITERS = EDGES_PER_TILE // CHUNK
PAIRS = FULL_ITERS // 2
TAIL = EDGES_PER_TILE - FULL_ITERS * CHUNK
ACC_ROWS = 10112
ROWS_PER_TILE = ACC_ROWS // 16
MLP_BLK = 3200
XW = 128


def _mlp_body(at_ref, sht_ref, w1_ref, b1_ref, w2_ref, b2_ref, o_ref):
    h = jnp.maximum(
        lax.dot_general(at_ref[...], w1_ref[...], (((0,), (0,)), ((), ())),
                        preferred_element_type=jnp.float32) + b1_ref[...], 0.0)
    w = jnp.dot(h, w2_ref[...], preferred_element_type=jnp.float32) + b2_ref[...]
    ws = w[:, :NS]
    wv = w[:, NS:NS + NV]
    er = lax.broadcasted_iota(jnp.int32, (4, 3), 0)
    ec = lax.broadcasted_iota(jnp.int32, (4, 3), 1)
    sel = (er == ec + 1).astype(jnp.float32)
    sh3 = lax.dot_general(sht_ref[...], sel, (((0,), (0,)), ((), ())),
                          preferred_element_type=jnp.float32)
    u0 = wv * sh3[:, 0:1]
    u1 = wv * sh3[:, 1:2]
    u2 = wv * sh3[:, 2:3]
    pad = jnp.zeros((MLP_BLK, MROW - WEXT), jnp.float32)
    o_ref[...] = jnp.concatenate([ws, u0, u1, u2, pad], axis=1)


def _mlp(edge_attr_t, edge_sh_t, W1, b1, W2, b2):
    grid = (N_EDGES // MLP_BLK,)
    return pl.pallas_call(
        _mlp_body,
        grid=grid,
        in_specs=[
            pl.BlockSpec((HID, MLP_BLK), lambda i: (0, i)),
            pl.BlockSpec((4, MLP_BLK), lambda i: (0, i)),
            pl.BlockSpec((HID, HID), lambda i: (0, 0)),
            pl.BlockSpec((1, HID), lambda i: (0, 0)),
            pl.BlockSpec((HID, NS + NV), lambda i: (0, 0)),
            pl.BlockSpec((1, NS + NV), lambda i: (0, 0)),
        ],
        out_specs=pl.BlockSpec((MLP_BLK, MROW), lambda i: (i, 0)),
        out_shape=jax.ShapeDtypeStruct((N_EDGES, MROW), jnp.float32),
    )(edge_attr_t, edge_sh_t, W1, b1, W2, b2)


def _sc_kernel(x, src, dst, wext):
    mesh = plsc.VectorSubcoreMesh(core_axis_name="c", subcore_axis_name="s")

    @functools.partial(
        pl.kernel,
        mesh=mesh,
        out_type=jax.ShapeDtypeStruct((2, ACC_ROWS, MROW), jnp.float32),
        scratch_types=[
            pltpu.VMEM((CHUNK,), jnp.int32),
            pltpu.VMEM((CHUNK,), jnp.int32),
            pltpu.VMEM((CHUNK,), jnp.int32),
            pltpu.VMEM((CHUNK,), jnp.int32),
            pltpu.VMEM((TAIL,), jnp.int32),
            pltpu.VMEM((TAIL,), jnp.int32),
            pltpu.VMEM((CHUNK, XW), jnp.float32),
            pltpu.VMEM((CHUNK, XW), jnp.float32),
            pltpu.VMEM((CHUNK, MROW), jnp.float32),
            pltpu.VMEM((CHUNK, MROW), jnp.float32),
            pltpu.VMEM((CHUNK, MROW), jnp.float32),
            pltpu.VMEM((CHUNK, MROW), jnp.float32),
            pltpu.VMEM_SHARED((ACC_ROWS, MROW), jnp.float32),
            pltpu.SemaphoreType.DMA,
            pltpu.SemaphoreType.DMA,
            pltpu.SemaphoreType.DMA,
            pltpu.SemaphoreType.DMA,
            pltpu.SemaphoreType.DMA,
            pltpu.SemaphoreType.DMA,
            pltpu.SemaphoreType.DMA,
            pltpu.SemaphoreType.DMA,
            pltpu.SemaphoreType.DMA,
        ],
    )
    def body(x_hbm, src_hbm, dst_hbm, w_hbm, out_hbm,
             idxs0, idxs1, idxd0, idxd1, idxs8, idxd8,
             hsrc0, hsrc1, w0, w1, msg0, msg1, acc_sh,
             sem_in0, sem_in1, sem_w0, sem_w1, sem_g0, sem_g1,
             sem_s0, sem_s1, sem):
        cid = lax.axis_index("c")
        sid = lax.axis_index("s")
        idxs = (idxs0, idxs1)
        idxd = (idxd0, idxd1)
        hsrc = (hsrc0, hsrc1)
        wv = (w0, w1)
        msg = (msg0, msg1)
        sem_in = (sem_in0, sem_in1)
        sem_w = (sem_w0, sem_w1)
        sem_g = (sem_g0, sem_g1)
        sem_s = (sem_s0, sem_s1)

        zeros16 = jnp.zeros((16,), jnp.float32)

        def zero_buf(buf):
            def f(i, _):
                r = i // (MROW // 16)
                c = i % (MROW // 16)
                buf[r, pl.ds(c * 16, 16)] = zeros16
                return 0
            lax.fori_loop(0, CHUNK * (MROW // 16), f, 0)

        zero_buf(msg0)
        zero_buf(msg1)

        for k in range(ROWS_PER_TILE // CHUNK):
            pltpu.sync_copy(
                msg0, acc_sh.at[pl.ds(sid * ROWS_PER_TILE + k * CHUNK, CHUNK)])
        rem = ROWS_PER_TILE % CHUNK
        if rem:
            pltpu.sync_copy(
                msg0.at[pl.ds(0, rem)],
                acc_sh.at[pl.ds(sid * ROWS_PER_TILE
                                + (ROWS_PER_TILE // CHUNK) * CHUNK, rem)])

        ii = lax.broadcasted_iota(jnp.int32, (16,), 0)

        cvec = jnp.where(ii == 0, 1.0, 0.0).astype(jnp.float32)

        def crow(buf):
            def f(r, _):
                buf[r, pl.ds(WEXT, 16)] = cvec
                return 0
            lax.fori_loop(0, CHUNK, f, 0)

        crow(msg0)
        crow(msg1)

        plsc.subcore_barrier()

        tbase = (cid * 16 + sid) * EDGES_PER_TILE

        def make_edge_body(hs, wb, mb):
            def edge_body(e, _):
                h0 = hs[e, pl.ds(0, 16)]
                mb[e, pl.ds(0, 16)] = h0 * wb[e, pl.ds(0, 16)]
                for k in range(1, NS // 16):
                    mb[e, pl.ds(k * 16, 16)] = (
                        hs[e, pl.ds(k * 16, 16)] * wb[e, pl.ds(k * 16, 16)])
                for j in range(3):
                    off = NS + j * 16
                    mb[e, pl.ds(off, 16)] = h0 * wb[e, pl.ds(off, 16)]
                return 0
            return edge_body

        def half_step(p, c, S, prefetch):
            pltpu.make_async_copy(x_hbm.at[idxs[S]], hsrc[S], sem_g[S]).wait()
            pltpu.make_async_copy(
                w_hbm.at[pl.ds(0, CHUNK)], wv[S], sem_w[S]).wait()

            @pl.when(p > 0)
            def _():
                pltpu.make_async_copy(
                    msg[S], acc_sh.at[idxd[S]], sem_s[S]).wait()

            base = tbase + c * CHUNK
            pltpu.sync_copy(dst_hbm.at[pl.ds(base, CHUNK)], idxd[S])

            @pl.when(prefetch)
            def _():
                nbase = tbase + (c + 2) * CHUNK
                pltpu.async_copy(
                    src_hbm.at[pl.ds(nbase, CHUNK)], idxs[S], sem_in[S])

            lax.fori_loop(0, CHUNK, make_edge_body(hsrc[S], wv[S], msg[S]), 0)

            pltpu.async_copy(msg[S], acc_sh.at[idxd[S]], sem_s[S], add=True)

            @pl.when(prefetch)
            def _():
                nbase = tbase + (c + 2) * CHUNK
                pltpu.async_copy(
                    w_hbm.at[pl.ds(nbase, CHUNK)], wv[S], sem_w[S])
                pltpu.make_async_copy(
                    src_hbm.at[pl.ds(nbase, CHUNK)], idxs[S], sem_in[S]).wait()
                pltpu.async_copy(x_hbm.at[idxs[S]], hsrc[S], sem_g[S])

        def pair_body(p, _):
            half_step(p, 2 * p, 0, p < PAIRS - 1)
            half_step(p, 2 * p + 1, 1, p < PAIRS - 1)
            return 0

        for S in range(2):
            base = tbase + S * CHUNK
            pltpu.async_copy(src_hbm.at[pl.ds(base, CHUNK)], idxs[S], sem_in[S])
            pltpu.async_copy(w_hbm.at[pl.ds(base, CHUNK)], wv[S], sem_w[S])
        for S in range(2):
            pltpu.make_async_copy(
                src_hbm.at[pl.ds(tbase, CHUNK)], idxs[S], sem_in[S]).wait()
            pltpu.async_copy(x_hbm.at[idxs[S]], hsrc[S], sem_g[S])

        lax.fori_loop(0, PAIRS, pair_body, 0)

        pltpu.make_async_copy(msg[0], acc_sh.at[idxd[0]], sem_s[0]).wait()
        pltpu.make_async_copy(msg[1], acc_sh.at[idxd[1]], sem_s[1]).wait()

        tb = tbase + FULL_ITERS * CHUNK
        pltpu.sync_copy(src_hbm.at[pl.ds(tb, TAIL)], idxs8)
        pltpu.sync_copy(dst_hbm.at[pl.ds(tb, TAIL)], idxd8)
        pltpu.sync_copy(w_hbm.at[pl.ds(tb, TAIL)], w0.at[pl.ds(0, TAIL)])
        pltpu.async_copy(x_hbm.at[idxs8], hsrc0.at[pl.ds(0, TAIL)], sem).wait()
        lax.fori_loop(0, TAIL, make_edge_body(hsrc0, w0, msg0), 0)
        pltpu.sync_copy(msg0.at[pl.ds(0, TAIL)], acc_sh.at[idxd8], add=True)

        plsc.subcore_barrier()

        pltpu.sync_copy(
            acc_sh.at[pl.ds(sid * ROWS_PER_TILE, ROWS_PER_TILE)],
            out_hbm.at[cid, pl.ds(sid * ROWS_PER_TILE, ROWS_PER_TILE)])

    return body(x, src, dst, wext)


def _combine_body(p_ref, o_ref):
    a = p_ref[0] + p_ref[1]
    deg = jnp.maximum(a[:, WEXT:WEXT + 1], 1.0)
    s = a[:, :NS] / deg
    v = a[:, NS:WEXT] / deg
    r = lax.broadcasted_iota(jnp.int32, (3 * NV, 3 * NV), 0)
    c = lax.broadcasted_iota(jnp.int32, (3 * NV, 3 * NV), 1)
    perm = ((c % 3) * NV + (c // 3) == r).astype(jnp.float32)
    vp = jnp.dot(v, perm, preferred_element_type=jnp.float32)
    o_ref[...] = jnp.concatenate([s, vp], axis=1)


def _combine(partials):
    grid = (10,)
    blk = N_NODES // 10
    return pl.pallas_call(
        _combine_body,
        grid=grid,
        in_specs=[pl.BlockSpec((2, blk, MROW), lambda i: (0, i, 0))],
        out_specs=pl.BlockSpec((blk, NS + 3 * NV), lambda i: (i, 0)),
        out_shape=jax.ShapeDtypeStruct((N_NODES, NS + 3 * NV), jnp.float32),
    )(partials)


def kernel(x, edge_index, edge_attr, edge_sh, W1, b1, W2, b2):
    src = edge_index[0].astype(jnp.int32)
    dst = edge_index[1].astype(jnp.int32)
    wext = _mlp(edge_attr.T, edge_sh.T, W1, b1.reshape(1, HID),
                W2, b2.reshape(1, NS + NV))
    x_pad = jnp.pad(x, ((0, 0), (0, XW - NS)))
    partials = _sc_kernel(x_pad, src, dst, wext)
    return _combine(partials)


# speedup vs baseline: 1.1637x; 1.0350x over previous
import functools

import jax
import jax.numpy as jnp
from jax import lax
from jax.experimental import pallas as pl
from jax.experimental.pallas import tpu as pltpu
from jax.experimental.pallas import tpu_sc as plsc

NS = 64
NV = 16
HID = 192
N_NODES = 10000
N_EDGES = 160000

WEXT = NS + 3 * NV
MROW = 128
NTILES = 32
CHUNK = 64
EDGES_PER_TILE = N_EDGES // NTILES
FULL_ITERS = EDGES_PER_TILE // CHUNK
PAIRS = FULL_ITERS // 2
TAIL = EDGES_PER_TILE - FULL_ITERS * CHUNK
ACC_ROWS = 10112
ROWS_PER_TILE = ACC_ROWS // 16
MLP_BLK = 3200
XW = 128


def _mlp_body(at_ref, sht_ref, w1_ref, b1_ref, w2_ref, b2_ref, o_ref):
    h = jnp.maximum(
        lax.dot_general(at_ref[...], w1_ref[...], (((0,), (0,)), ((), ())),
                        preferred_element_type=jnp.float32) + b1_ref[...], 0.0)
    w = jnp.dot(h, w2_ref[...], preferred_element_type=jnp.float32) + b2_ref[...]
    ws = w[:, :NS]
    wv = w[:, NS:NS + NV]
    er = lax.broadcasted_iota(jnp.int32, (4, 3), 0)
    ec = lax.broadcasted_iota(jnp.int32, (4, 3), 1)
    sel = (er == ec + 1).astype(jnp.float32)
    sh3 = lax.dot_general(sht_ref[...], sel, (((0,), (0,)), ((), ())),
                          preferred_element_type=jnp.float32)
    u0 = wv * sh3[:, 0:1]
    u1 = wv * sh3[:, 1:2]
    u2 = wv * sh3[:, 2:3]
    pad = jnp.zeros((MLP_BLK, MROW - WEXT), jnp.float32)
    o_ref[...] = jnp.concatenate([ws, u0, u1, u2, pad], axis=1)


def _mlp(edge_attr_t, edge_sh_t, W1, b1, W2, b2):
    grid = (N_EDGES // MLP_BLK,)
    return pl.pallas_call(
        _mlp_body,
        grid=grid,
        in_specs=[
            pl.BlockSpec((HID, MLP_BLK), lambda i: (0, i)),
            pl.BlockSpec((4, MLP_BLK), lambda i: (0, i)),
            pl.BlockSpec((HID, HID), lambda i: (0, 0)),
            pl.BlockSpec((1, HID), lambda i: (0, 0)),
            pl.BlockSpec((HID, NS + NV), lambda i: (0, 0)),
            pl.BlockSpec((1, NS + NV), lambda i: (0, 0)),
        ],
        out_specs=pl.BlockSpec((MLP_BLK, MROW), lambda i: (i, 0)),
        out_shape=jax.ShapeDtypeStruct((N_EDGES, MROW), jnp.float32),
    )(edge_attr_t, edge_sh_t, W1, b1, W2, b2)


def _sc_kernel(x, src, dst, wext):
    mesh = plsc.VectorSubcoreMesh(core_axis_name="c", subcore_axis_name="s")

    @functools.partial(
        pl.kernel,
        mesh=mesh,
        out_type=jax.ShapeDtypeStruct((2, ACC_ROWS, MROW), jnp.float32),
        scratch_types=[
            pltpu.VMEM((CHUNK,), jnp.int32),
            pltpu.VMEM((CHUNK,), jnp.int32),
            pltpu.VMEM((CHUNK,), jnp.int32),
            pltpu.VMEM((CHUNK,), jnp.int32),
            pltpu.VMEM((TAIL,), jnp.int32),
            pltpu.VMEM((TAIL,), jnp.int32),
            pltpu.VMEM((CHUNK, XW), jnp.float32),
            pltpu.VMEM((CHUNK, XW), jnp.float32),
            pltpu.VMEM((CHUNK, MROW), jnp.float32),
            pltpu.VMEM((CHUNK, MROW), jnp.float32),
            pltpu.VMEM((CHUNK, MROW), jnp.float32),
            pltpu.VMEM((CHUNK, MROW), jnp.float32),
            pltpu.VMEM_SHARED((ACC_ROWS, MROW), jnp.float32),
            pltpu.SemaphoreType.DMA,
            pltpu.SemaphoreType.DMA,
            pltpu.SemaphoreType.DMA,
            pltpu.SemaphoreType.DMA,
            pltpu.SemaphoreType.DMA,
            pltpu.SemaphoreType.DMA,
            pltpu.SemaphoreType.DMA,
            pltpu.SemaphoreType.DMA,
            pltpu.SemaphoreType.DMA,
        ],
    )
    def body(x_hbm, src_hbm, dst_hbm, w_hbm, out_hbm,
             idxs0, idxs1, idxd0, idxd1, idxs8, idxd8,
             hsrc0, hsrc1, w0, w1, msg0, msg1, acc_sh,
             sem_in0, sem_in1, sem_w0, sem_w1, sem_g0, sem_g1,
             sem_s0, sem_s1, sem):
        cid = lax.axis_index("c")
        sid = lax.axis_index("s")
        idxs = (idxs0, idxs1)
        idxd = (idxd0, idxd1)
        hsrc = (hsrc0, hsrc1)
        wv = (w0, w1)
        msg = (msg0, msg1)
        sem_in = (sem_in0, sem_in1)
        sem_w = (sem_w0, sem_w1)
        sem_g = (sem_g0, sem_g1)
        sem_s = (sem_s0, sem_s1)

        zeros16 = jnp.zeros((16,), jnp.float32)

        def zero_buf(buf):
            def f(i, _):
                r = i // (MROW // 16)
                c = i % (MROW // 16)
                buf[r, pl.ds(c * 16, 16)] = zeros16
                return 0
            lax.fori_loop(0, CHUNK * (MROW // 16), f, 0)

        zero_buf(msg0)
        zero_buf(msg1)

        for k in range(ROWS_PER_TILE // CHUNK):
            pltpu.sync_copy(
                msg0, acc_sh.at[pl.ds(sid * ROWS_PER_TILE + k * CHUNK, CHUNK)])
        rem = ROWS_PER_TILE % CHUNK
        if rem:
            pltpu.sync_copy(
                msg0.at[pl.ds(0, rem)],
                acc_sh.at[pl.ds(sid * ROWS_PER_TILE
                                + (ROWS_PER_TILE // CHUNK) * CHUNK, rem)])

        ii = lax.broadcasted_iota(jnp.int32, (16,), 0)

        cvec = jnp.where(ii == 0, 1.0, 0.0).astype(jnp.float32)

        def crow(buf):
            def f(r, _):
                buf[r, pl.ds(WEXT, 16)] = cvec
                return 0
            lax.fori_loop(0, CHUNK, f, 0)

        crow(msg0)
        crow(msg1)

        plsc.subcore_barrier()

        tbase = (cid * 16 + sid) * EDGES_PER_TILE

        def make_edge_body(hs, wb, mb):
            def edge_body(e, _):
                h0 = hs[e, pl.ds(0, 16)]
                mb[e, pl.ds(0, 16)] = h0 * wb[e, pl.ds(0, 16)]
                for k in range(1, NS // 16):
                    mb[e, pl.ds(k * 16, 16)] = (
                        hs[e, pl.ds(k * 16, 16)] * wb[e, pl.ds(k * 16, 16)])
                for j in range(3):
                    off = NS + j * 16
                    mb[e, pl.ds(off, 16)] = h0 * wb[e, pl.ds(off, 16)]
                return 0
            return edge_body

        def half_step(p, c, S, prefetch):
            pltpu.make_async_copy(x_hbm.at[idxs[S]], hsrc[S], sem_g[S]).wait()
            pltpu.make_async_copy(
                w_hbm.at[pl.ds(0, CHUNK)], wv[S], sem_w[S]).wait()

            @pl.when(p > 0)
            def _():
                pltpu.make_async_copy(
                    msg[S], acc_sh.at[idxd[S]], sem_s[S]).wait()

            base = tbase + c * CHUNK
            pltpu.async_copy(dst_hbm.at[pl.ds(base, CHUNK)], idxd[S], sem)

            @pl.when(prefetch)
            def _():
                nbase = tbase + (c + 2) * CHUNK
                pltpu.async_copy(
                    src_hbm.at[pl.ds(nbase, CHUNK)], idxs[S], sem_in[S])

            lax.fori_loop(0, CHUNK, make_edge_body(hsrc[S], wv[S], msg[S]), 0)

            pltpu.make_async_copy(
                dst_hbm.at[pl.ds(base, CHUNK)], idxd[S], sem).wait()
            pltpu.async_copy(msg[S], acc_sh.at[idxd[S]], sem_s[S], add=True)

            @pl.when(prefetch)
            def _():
                nbase = tbase + (c + 2) * CHUNK
                pltpu.async_copy(
                    w_hbm.at[pl.ds(nbase, CHUNK)], wv[S], sem_w[S])
                pltpu.make_async_copy(
                    src_hbm.at[pl.ds(nbase, CHUNK)], idxs[S], sem_in[S]).wait()
                pltpu.async_copy(x_hbm.at[idxs[S]], hsrc[S], sem_g[S])

        def pair_body(p, _):
            half_step(p, 2 * p, 0, p < PAIRS - 1)
            half_step(p, 2 * p + 1, 1, p < PAIRS - 1)
            return 0

        for S in range(2):
            base = tbase + S * CHUNK
            pltpu.async_copy(src_hbm.at[pl.ds(base, CHUNK)], idxs[S], sem_in[S])
            pltpu.async_copy(w_hbm.at[pl.ds(base, CHUNK)], wv[S], sem_w[S])
        for S in range(2):
            pltpu.make_async_copy(
                src_hbm.at[pl.ds(tbase, CHUNK)], idxs[S], sem_in[S]).wait()
            pltpu.async_copy(x_hbm.at[idxs[S]], hsrc[S], sem_g[S])

        lax.fori_loop(0, PAIRS, pair_body, 0)

        pltpu.make_async_copy(msg[0], acc_sh.at[idxd[0]], sem_s[0]).wait()
        pltpu.make_async_copy(msg[1], acc_sh.at[idxd[1]], sem_s[1]).wait()

        tb = tbase + FULL_ITERS * CHUNK
        pltpu.sync_copy(src_hbm.at[pl.ds(tb, TAIL)], idxs8)
        pltpu.sync_copy(dst_hbm.at[pl.ds(tb, TAIL)], idxd8)
        pltpu.sync_copy(w_hbm.at[pl.ds(tb, TAIL)], w0.at[pl.ds(0, TAIL)])
        pltpu.async_copy(x_hbm.at[idxs8], hsrc0.at[pl.ds(0, TAIL)], sem).wait()
        lax.fori_loop(0, TAIL, make_edge_body(hsrc0, w0, msg0), 0)
        pltpu.sync_copy(msg0.at[pl.ds(0, TAIL)], acc_sh.at[idxd8], add=True)

        plsc.subcore_barrier()

        pltpu.sync_copy(
            acc_sh.at[pl.ds(sid * ROWS_PER_TILE, ROWS_PER_TILE)],
            out_hbm.at[cid, pl.ds(sid * ROWS_PER_TILE, ROWS_PER_TILE)])

    return body(x, src, dst, wext)


def _combine_body(p_ref, o_ref):
    a = p_ref[0] + p_ref[1]
    deg = jnp.maximum(a[:, WEXT:WEXT + 1], 1.0)
    s = a[:, :NS] / deg
    v = a[:, NS:WEXT] / deg
    r = lax.broadcasted_iota(jnp.int32, (3 * NV, 3 * NV), 0)
    c = lax.broadcasted_iota(jnp.int32, (3 * NV, 3 * NV), 1)
    perm = ((c % 3) * NV + (c // 3) == r).astype(jnp.float32)
    vp = jnp.dot(v, perm, preferred_element_type=jnp.float32)
    o_ref[...] = jnp.concatenate([s, vp], axis=1)


def _combine(partials):
    grid = (10,)
    blk = N_NODES // 10
    return pl.pallas_call(
        _combine_body,
        grid=grid,
        in_specs=[pl.BlockSpec((2, blk, MROW), lambda i: (0, i, 0))],
        out_specs=pl.BlockSpec((blk, NS + 3 * NV), lambda i: (i, 0)),
        out_shape=jax.ShapeDtypeStruct((N_NODES, NS + 3 * NV), jnp.float32),
    )(partials)


def kernel(x, edge_index, edge_attr, edge_sh, W1, b1, W2, b2):
    src = edge_index[0].astype(jnp.int32)
    dst = edge_index[1].astype(jnp.int32)
    wext = _mlp(edge_attr.T, edge_sh.T, W1, b1.reshape(1, HID),
                W2, b2.reshape(1, NS + NV))
    x_pad = jnp.pad(x, ((0, 0), (0, XW - NS)))
    partials = _sc_kernel(x_pad, src, dst, wext)
    return _combine(partials)


# speedup vs baseline: 1.2853x; 1.1045x over previous
import functools

import jax
import jax.numpy as jnp
from jax import lax
from jax.experimental import pallas as pl
from jax.experimental.pallas import tpu as pltpu
from jax.experimental.pallas import tpu_sc as plsc

NS = 64
NV = 16
HID = 192
N_NODES = 10000
N_EDGES = 160000

WEXT = NS + 3 * NV
MROW = 128
NTILES = 32
CHUNK = 64
ACC_ROWS = 10112
ROWS_PER_TILE = ACC_ROWS // 16
MLP_BLK = 3200
XW = 128
SPLIT = 102400


def _mlp_body(at_ref, sht_ref, w1_ref, b1_ref, w2_ref, b2_ref, o_ref):
    h = jnp.maximum(
        lax.dot_general(at_ref[...], w1_ref[...], (((0,), (0,)), ((), ())),
                        preferred_element_type=jnp.float32) + b1_ref[...], 0.0)
    w = jnp.dot(h, w2_ref[...], preferred_element_type=jnp.float32) + b2_ref[...]
    ws = w[:, :NS]
    wv = w[:, NS:NS + NV]
    er = lax.broadcasted_iota(jnp.int32, (4, 3), 0)
    ec = lax.broadcasted_iota(jnp.int32, (4, 3), 1)
    sel = (er == ec + 1).astype(jnp.float32)
    sh3 = lax.dot_general(sht_ref[...], sel, (((0,), (0,)), ((), ())),
                          preferred_element_type=jnp.float32)
    u0 = wv * sh3[:, 0:1]
    u1 = wv * sh3[:, 1:2]
    u2 = wv * sh3[:, 2:3]
    pad = jnp.zeros((MLP_BLK, MROW - WEXT), jnp.float32)
    o_ref[...] = jnp.concatenate([ws, u0, u1, u2, pad], axis=1)


def _mlp(edge_attr_t, edge_sh_t, W1, b1, W2, b2, n_edges, blk_off):
    grid = (n_edges // MLP_BLK,)
    return pl.pallas_call(
        _mlp_body,
        grid=grid,
        in_specs=[
            pl.BlockSpec((HID, MLP_BLK), lambda i: (0, i + blk_off)),
            pl.BlockSpec((4, MLP_BLK), lambda i: (0, i + blk_off)),
            pl.BlockSpec((HID, HID), lambda i: (0, 0)),
            pl.BlockSpec((1, HID), lambda i: (0, 0)),
            pl.BlockSpec((HID, NS + NV), lambda i: (0, 0)),
            pl.BlockSpec((1, NS + NV), lambda i: (0, 0)),
        ],
        out_specs=pl.BlockSpec((MLP_BLK, MROW), lambda i: (i, 0)),
        out_shape=jax.ShapeDtypeStruct((n_edges, MROW), jnp.float32),
    )(edge_attr_t, edge_sh_t, W1, b1, W2, b2)


def _sc_kernel(x, src, dst, wext, ebase, n_edges):
    ept = n_edges // NTILES
    full_iters = ept // CHUNK
    pairs = full_iters // 2
    tail = ept - full_iters * CHUNK
    assert full_iters % 2 == 0 and ept % 8 == 0 and (tail == 0 or tail >= 8)
    mesh = plsc.VectorSubcoreMesh(core_axis_name="c", subcore_axis_name="s")

    @functools.partial(
        pl.kernel,
        mesh=mesh,
        out_type=jax.ShapeDtypeStruct((2, ACC_ROWS, MROW), jnp.float32),
        scratch_types=[
            pltpu.VMEM((CHUNK,), jnp.int32),
            pltpu.VMEM((CHUNK,), jnp.int32),
            pltpu.VMEM((CHUNK,), jnp.int32),
            pltpu.VMEM((CHUNK,), jnp.int32),
            pltpu.VMEM((8,), jnp.int32),
            pltpu.VMEM((8,), jnp.int32),
            pltpu.VMEM((CHUNK, XW), jnp.float32),
            pltpu.VMEM((CHUNK, XW), jnp.float32),
            pltpu.VMEM((CHUNK, MROW), jnp.float32),
            pltpu.VMEM((CHUNK, MROW), jnp.float32),
            pltpu.VMEM((CHUNK, MROW), jnp.float32),
            pltpu.VMEM((CHUNK, MROW), jnp.float32),
            pltpu.VMEM_SHARED((ACC_ROWS, MROW), jnp.float32),
            pltpu.SemaphoreType.DMA,
            pltpu.SemaphoreType.DMA,
            pltpu.SemaphoreType.DMA,
            pltpu.SemaphoreType.DMA,
            pltpu.SemaphoreType.DMA,
            pltpu.SemaphoreType.DMA,
            pltpu.SemaphoreType.DMA,
            pltpu.SemaphoreType.DMA,
            pltpu.SemaphoreType.DMA,
        ],
    )
    def body(x_hbm, src_hbm, dst_hbm, w_hbm, out_hbm,
             idxs0, idxs1, idxd0, idxd1, idxs8, idxd8,
             hsrc0, hsrc1, w0, w1, msg0, msg1, acc_sh,
             sem_in0, sem_in1, sem_w0, sem_w1, sem_g0, sem_g1,
             sem_s0, sem_s1, sem):
        cid = lax.axis_index("c")
        sid = lax.axis_index("s")
        idxs = (idxs0, idxs1)
        idxd = (idxd0, idxd1)
        hsrc = (hsrc0, hsrc1)
        wv = (w0, w1)
        msg = (msg0, msg1)
        sem_in = (sem_in0, sem_in1)
        sem_w = (sem_w0, sem_w1)
        sem_g = (sem_g0, sem_g1)
        sem_s = (sem_s0, sem_s1)

        zeros16 = jnp.zeros((16,), jnp.float32)

        def zero_buf(buf):
            def f(i, _):
                r = i // (MROW // 16)
                c = i % (MROW // 16)
                buf[r, pl.ds(c * 16, 16)] = zeros16
                return 0
            lax.fori_loop(0, CHUNK * (MROW // 16), f, 0)

        zero_buf(msg0)
        zero_buf(msg1)

        for k in range(ROWS_PER_TILE // CHUNK):
            pltpu.sync_copy(
                msg0, acc_sh.at[pl.ds(sid * ROWS_PER_TILE + k * CHUNK, CHUNK)])
        rem = ROWS_PER_TILE % CHUNK
        if rem:
            pltpu.sync_copy(
                msg0.at[pl.ds(0, rem)],
                acc_sh.at[pl.ds(sid * ROWS_PER_TILE
                                + (ROWS_PER_TILE // CHUNK) * CHUNK, rem)])

        ii = lax.broadcasted_iota(jnp.int32, (16,), 0)
        cvec = jnp.where(ii == 0, 1.0, 0.0).astype(jnp.float32)

        def crow(buf):
            def f(r, _):
                buf[r, pl.ds(WEXT, 16)] = cvec
                return 0
            lax.fori_loop(0, CHUNK, f, 0)

        crow(msg0)
        crow(msg1)

        plsc.subcore_barrier()

        tloc = (cid * 16 + sid) * ept
        tbase = ebase + tloc

        def make_edge_body(hs, wb, mb):
            def edge_body(e, _):
                h0 = hs[e, pl.ds(0, 16)]
                mb[e, pl.ds(0, 16)] = h0 * wb[e, pl.ds(0, 16)]
                for k in range(1, NS // 16):
                    mb[e, pl.ds(k * 16, 16)] = (
                        hs[e, pl.ds(k * 16, 16)] * wb[e, pl.ds(k * 16, 16)])
                for j in range(3):
                    off = NS + j * 16
                    mb[e, pl.ds(off, 16)] = h0 * wb[e, pl.ds(off, 16)]
                return 0
            return edge_body

        def half_step(p, c, S, prefetch):
            pltpu.make_async_copy(x_hbm.at[idxs[S]], hsrc[S], sem_g[S]).wait()
            pltpu.make_async_copy(
                w_hbm.at[pl.ds(0, CHUNK)], wv[S], sem_w[S]).wait()

            @pl.when(p > 0)
            def _():
                pltpu.make_async_copy(
                    msg[S], acc_sh.at[idxd[S]], sem_s[S]).wait()

            base = tbase + c * CHUNK
            pltpu.async_copy(dst_hbm.at[pl.ds(base, CHUNK)], idxd[S], sem)

            @pl.when(prefetch)
            def _():
                nbase = tbase + (c + 2) * CHUNK
                pltpu.async_copy(
                    src_hbm.at[pl.ds(nbase, CHUNK)], idxs[S], sem_in[S])

            lax.fori_loop(0, CHUNK, make_edge_body(hsrc[S], wv[S], msg[S]), 0)

            pltpu.make_async_copy(
                dst_hbm.at[pl.ds(base, CHUNK)], idxd[S], sem).wait()
            pltpu.async_copy(msg[S], acc_sh.at[idxd[S]], sem_s[S], add=True)

            @pl.when(prefetch)
            def _():
                nloc = tloc + (c + 2) * CHUNK
                nbase = tbase + (c + 2) * CHUNK
                pltpu.async_copy(
                    w_hbm.at[pl.ds(nloc, CHUNK)], wv[S], sem_w[S])
                pltpu.make_async_copy(
                    src_hbm.at[pl.ds(nbase, CHUNK)], idxs[S], sem_in[S]).wait()
                pltpu.async_copy(x_hbm.at[idxs[S]], hsrc[S], sem_g[S])

        def pair_body(p, _):
            half_step(p, 2 * p, 0, p < pairs - 1)
            half_step(p, 2 * p + 1, 1, p < pairs - 1)
            return 0

        for S in range(2):
            pltpu.async_copy(
                src_hbm.at[pl.ds(tbase + S * CHUNK, CHUNK)], idxs[S], sem_in[S])
            pltpu.async_copy(
                w_hbm.at[pl.ds(tloc + S * CHUNK, CHUNK)], wv[S], sem_w[S])
        for S in range(2):
            pltpu.make_async_copy(
                src_hbm.at[pl.ds(tbase, CHUNK)], idxs[S], sem_in[S]).wait()
            pltpu.async_copy(x_hbm.at[idxs[S]], hsrc[S], sem_g[S])

        lax.fori_loop(0, pairs, pair_body, 0)

        pltpu.make_async_copy(msg[0], acc_sh.at[idxd[0]], sem_s[0]).wait()
        pltpu.make_async_copy(msg[1], acc_sh.at[idxd[1]], sem_s[1]).wait()

        if tail:
            tl = tloc + full_iters * CHUNK
            tb = tbase + full_iters * CHUNK
            pltpu.sync_copy(src_hbm.at[pl.ds(tb, tail)], idxs8)
            pltpu.sync_copy(dst_hbm.at[pl.ds(tb, tail)], idxd8)
            pltpu.sync_copy(w_hbm.at[pl.ds(tl, tail)], w0.at[pl.ds(0, tail)])
            pltpu.async_copy(
                x_hbm.at[idxs8], hsrc0.at[pl.ds(0, tail)], sem).wait()
            lax.fori_loop(0, tail, make_edge_body(hsrc0, w0, msg0), 0)
            pltpu.sync_copy(msg0.at[pl.ds(0, tail)], acc_sh.at[idxd8], add=True)

        plsc.subcore_barrier()

        pltpu.sync_copy(
            acc_sh.at[pl.ds(sid * ROWS_PER_TILE, ROWS_PER_TILE)],
            out_hbm.at[cid, pl.ds(sid * ROWS_PER_TILE, ROWS_PER_TILE)])

    return body(x, src, dst, wext)


def _combine_body(pa_ref, pb_ref, o_ref):
    a = pa_ref[0] + pa_ref[1] + pb_ref[0] + pb_ref[1]
    deg = jnp.maximum(a[:, WEXT:WEXT + 1], 1.0)
    s = a[:, :NS] / deg
    v = a[:, NS:WEXT] / deg
    r = lax.broadcasted_iota(jnp.int32, (3 * NV, 3 * NV), 0)
    c = lax.broadcasted_iota(jnp.int32, (3 * NV, 3 * NV), 1)
    perm = ((c % 3) * NV + (c // 3) == r).astype(jnp.float32)
    vp = jnp.dot(v, perm, preferred_element_type=jnp.float32)
    o_ref[...] = jnp.concatenate([s, vp], axis=1)


def _combine(pa, pb):
    grid = (10,)
    blk = N_NODES // 10
    spec = pl.BlockSpec((2, blk, MROW), lambda i: (0, i, 0))
    return pl.pallas_call(
        _combine_body,
        grid=grid,
        in_specs=[spec, spec],
        out_specs=pl.BlockSpec((blk, NS + 3 * NV), lambda i: (i, 0)),
        out_shape=jax.ShapeDtypeStruct((N_NODES, NS + 3 * NV), jnp.float32),
    )(pa, pb)


def kernel(x, edge_index, edge_attr, edge_sh, W1, b1, W2, b2):
    src = edge_index[0].astype(jnp.int32)
    dst = edge_index[1].astype(jnp.int32)
    ea_t = edge_attr.T
    sh_t = edge_sh.T
    b1r = b1.reshape(1, HID)
    b2r = b2.reshape(1, NS + NV)
    x_pad = jnp.pad(x, ((0, 0), (0, XW - NS)))

    wext_a = _mlp(ea_t, sh_t, W1, b1r, W2, b2r, SPLIT, 0)
    pa = _sc_kernel(x_pad, src, dst, wext_a, 0, SPLIT)
    wext_b = _mlp(ea_t, sh_t, W1, b1r, W2, b2r, N_EDGES - SPLIT,
                  SPLIT // MLP_BLK)
    pb = _sc_kernel(x_pad, src, dst, wext_b, SPLIT, N_EDGES - SPLIT)
    return _combine(pa, pb)


# speedup vs baseline: 1.3264x; 1.0320x over previous
import functools

import jax
import jax.numpy as jnp
from jax import lax
from jax.experimental import pallas as pl
from jax.experimental.pallas import tpu as pltpu
from jax.experimental.pallas import tpu_sc as plsc

NS = 64
NV = 16
HID = 192
N_NODES = 10000
N_EDGES = 160000

WEXT = NS + 3 * NV
MROW = 128
NTILES = 32
CHUNK = 64
ACC_ROWS = 10112
ROWS_PER_TILE = ACC_ROWS // 16
MLP_BLK = 3200
XW = 128
SPLIT = 89600


def _mlp_body(at_ref, sht_ref, w1_ref, b1_ref, w2_ref, b2_ref, o_ref):
    h = jnp.maximum(
        lax.dot_general(at_ref[...], w1_ref[...], (((0,), (0,)), ((), ())),
                        preferred_element_type=jnp.float32) + b1_ref[...], 0.0)
    w = jnp.dot(h, w2_ref[...], preferred_element_type=jnp.float32) + b2_ref[...]
    ws = w[:, :NS]
    wv = w[:, NS:NS + NV]
    er = lax.broadcasted_iota(jnp.int32, (4, 3), 0)
    ec = lax.broadcasted_iota(jnp.int32, (4, 3), 1)
    sel = (er == ec + 1).astype(jnp.float32)
    sh3 = lax.dot_general(sht_ref[...], sel, (((0,), (0,)), ((), ())),
                          preferred_element_type=jnp.float32)
    u0 = wv * sh3[:, 0:1]
    u1 = wv * sh3[:, 1:2]
    u2 = wv * sh3[:, 2:3]
    pad = jnp.zeros((MLP_BLK, MROW - WEXT), jnp.float32)
    o_ref[...] = jnp.concatenate([ws, u0, u1, u2, pad], axis=1)


def _mlp(edge_attr_t, edge_sh_t, W1, b1, W2, b2, n_edges, blk_off):
    grid = (n_edges // MLP_BLK,)
    return pl.pallas_call(
        _mlp_body,
        grid=grid,
        in_specs=[
            pl.BlockSpec((HID, MLP_BLK), lambda i: (0, i + blk_off)),
            pl.BlockSpec((4, MLP_BLK), lambda i: (0, i + blk_off)),
            pl.BlockSpec((HID, HID), lambda i: (0, 0)),
            pl.BlockSpec((1, HID), lambda i: (0, 0)),
            pl.BlockSpec((HID, NS + NV), lambda i: (0, 0)),
            pl.BlockSpec((1, NS + NV), lambda i: (0, 0)),
        ],
        out_specs=pl.BlockSpec((MLP_BLK, MROW), lambda i: (i, 0)),
        out_shape=jax.ShapeDtypeStruct((n_edges, MROW), jnp.float32),
    )(edge_attr_t, edge_sh_t, W1, b1, W2, b2)


def _sc_kernel(x, src, dst, wext, ebase, n_edges):
    ept = n_edges // NTILES
    full_iters = ept // CHUNK
    pairs = full_iters // 2
    epi = full_iters - 2 * pairs
    tail = ept - full_iters * CHUNK
    assert ept % 8 == 0 and (tail == 0 or tail >= 8) and pairs >= 2
    mesh = plsc.VectorSubcoreMesh(core_axis_name="c", subcore_axis_name="s")

    @functools.partial(
        pl.kernel,
        mesh=mesh,
        out_type=jax.ShapeDtypeStruct((2, ACC_ROWS, MROW), jnp.float32),
        scratch_types=[
            pltpu.VMEM((CHUNK,), jnp.int32),
            pltpu.VMEM((CHUNK,), jnp.int32),
            pltpu.VMEM((CHUNK,), jnp.int32),
            pltpu.VMEM((CHUNK,), jnp.int32),
            pltpu.VMEM((max(tail, 8),), jnp.int32),
            pltpu.VMEM((max(tail, 8),), jnp.int32),
            pltpu.VMEM((CHUNK, XW), jnp.float32),
            pltpu.VMEM((CHUNK, XW), jnp.float32),
            pltpu.VMEM((CHUNK, MROW), jnp.float32),
            pltpu.VMEM((CHUNK, MROW), jnp.float32),
            pltpu.VMEM((CHUNK, MROW), jnp.float32),
            pltpu.VMEM((CHUNK, MROW), jnp.float32),
            pltpu.VMEM_SHARED((ACC_ROWS, MROW), jnp.float32),
            pltpu.SemaphoreType.DMA,
            pltpu.SemaphoreType.DMA,
            pltpu.SemaphoreType.DMA,
            pltpu.SemaphoreType.DMA,
            pltpu.SemaphoreType.DMA,
            pltpu.SemaphoreType.DMA,
            pltpu.SemaphoreType.DMA,
            pltpu.SemaphoreType.DMA,
            pltpu.SemaphoreType.DMA,
        ],
    )
    def body(x_hbm, src_hbm, dst_hbm, w_hbm, out_hbm,
             idxs0, idxs1, idxd0, idxd1, idxs8, idxd8,
             hsrc0, hsrc1, w0, w1, msg0, msg1, acc_sh,
             sem_in0, sem_in1, sem_w0, sem_w1, sem_g0, sem_g1,
             sem_s0, sem_s1, sem):
        cid = lax.axis_index("c")
        sid = lax.axis_index("s")
        idxs = (idxs0, idxs1)
        idxd = (idxd0, idxd1)
        hsrc = (hsrc0, hsrc1)
        wv = (w0, w1)
        msg = (msg0, msg1)
        sem_in = (sem_in0, sem_in1)
        sem_w = (sem_w0, sem_w1)
        sem_g = (sem_g0, sem_g1)
        sem_s = (sem_s0, sem_s1)

        zeros16 = jnp.zeros((16,), jnp.float32)

        def zero_buf(buf):
            def f(i, _):
                r = i // (MROW // 16)
                c = i % (MROW // 16)
                buf[r, pl.ds(c * 16, 16)] = zeros16
                return 0
            lax.fori_loop(0, CHUNK * (MROW // 16), f, 0)

        zero_buf(msg0)
        zero_buf(msg1)

        for k in range(ROWS_PER_TILE // CHUNK):
            pltpu.sync_copy(
                msg0, acc_sh.at[pl.ds(sid * ROWS_PER_TILE + k * CHUNK, CHUNK)])
        rem = ROWS_PER_TILE % CHUNK
        if rem:
            pltpu.sync_copy(
                msg0.at[pl.ds(0, rem)],
                acc_sh.at[pl.ds(sid * ROWS_PER_TILE
                                + (ROWS_PER_TILE // CHUNK) * CHUNK, rem)])

        ii = lax.broadcasted_iota(jnp.int32, (16,), 0)
        cvec = jnp.where(ii == 0, 1.0, 0.0).astype(jnp.float32)

        def crow(buf):
            def f(r, _):
                buf[r, pl.ds(WEXT, 16)] = cvec
                return 0
            lax.fori_loop(0, CHUNK, f, 0)

        crow(msg0)
        crow(msg1)

        plsc.subcore_barrier()

        tloc = (cid * 16 + sid) * ept
        tbase = ebase + tloc

        def make_edge_body(hs, wb, mb):
            def edge_body(e, _):
                h0 = hs[e, pl.ds(0, 16)]
                mb[e, pl.ds(0, 16)] = h0 * wb[e, pl.ds(0, 16)]
                for k in range(1, NS // 16):
                    mb[e, pl.ds(k * 16, 16)] = (
                        hs[e, pl.ds(k * 16, 16)] * wb[e, pl.ds(k * 16, 16)])
                for j in range(3):
                    off = NS + j * 16
                    mb[e, pl.ds(off, 16)] = h0 * wb[e, pl.ds(off, 16)]
                return 0
            return edge_body

        def half_step(p, c, S, prefetch):
            pltpu.make_async_copy(x_hbm.at[idxs[S]], hsrc[S], sem_g[S]).wait()
            pltpu.make_async_copy(
                w_hbm.at[pl.ds(0, CHUNK)], wv[S], sem_w[S]).wait()

            @pl.when(p > 0)
            def _():
                pltpu.make_async_copy(
                    msg[S], acc_sh.at[idxd[S]], sem_s[S]).wait()

            base = tbase + c * CHUNK
            pltpu.async_copy(dst_hbm.at[pl.ds(base, CHUNK)], idxd[S], sem)

            @pl.when(prefetch)
            def _():
                nbase = tbase + (c + 2) * CHUNK
                pltpu.async_copy(
                    src_hbm.at[pl.ds(nbase, CHUNK)], idxs[S], sem_in[S])

            lax.fori_loop(0, CHUNK, make_edge_body(hsrc[S], wv[S], msg[S]), 0)

            pltpu.make_async_copy(
                dst_hbm.at[pl.ds(base, CHUNK)], idxd[S], sem).wait()
            pltpu.async_copy(msg[S], acc_sh.at[idxd[S]], sem_s[S], add=True)

            @pl.when(prefetch)
            def _():
                nloc = tloc + (c + 2) * CHUNK
                nbase = tbase + (c + 2) * CHUNK
                pltpu.async_copy(
                    w_hbm.at[pl.ds(nloc, CHUNK)], wv[S], sem_w[S])
                pltpu.make_async_copy(
                    src_hbm.at[pl.ds(nbase, CHUNK)], idxs[S], sem_in[S]).wait()
                pltpu.async_copy(x_hbm.at[idxs[S]], hsrc[S], sem_g[S])

        def pair_body(p, _):
            half_step(p, 2 * p, 0, p < pairs - 1 + epi)
            half_step(p, 2 * p + 1, 1, p < pairs - 1)
            return 0

        for S in range(2):
            pltpu.async_copy(
                src_hbm.at[pl.ds(tbase + S * CHUNK, CHUNK)], idxs[S], sem_in[S])
            pltpu.async_copy(
                w_hbm.at[pl.ds(tloc + S * CHUNK, CHUNK)], wv[S], sem_w[S])
        for S in range(2):
            pltpu.make_async_copy(
                src_hbm.at[pl.ds(tbase, CHUNK)], idxs[S], sem_in[S]).wait()
            pltpu.async_copy(x_hbm.at[idxs[S]], hsrc[S], sem_g[S])

        lax.fori_loop(0, pairs, pair_body, 0)

        if epi:
            ce = 2 * pairs
            pltpu.make_async_copy(x_hbm.at[idxs[0]], hsrc[0], sem_g[0]).wait()
            pltpu.make_async_copy(
                w_hbm.at[pl.ds(0, CHUNK)], wv[0], sem_w[0]).wait()
            pltpu.make_async_copy(msg[0], acc_sh.at[idxd[0]], sem_s[0]).wait()
            ebch = tbase + ce * CHUNK
            pltpu.async_copy(dst_hbm.at[pl.ds(ebch, CHUNK)], idxd[0], sem)
            lax.fori_loop(0, CHUNK, make_edge_body(hsrc[0], wv[0], msg[0]), 0)
            pltpu.make_async_copy(
                dst_hbm.at[pl.ds(ebch, CHUNK)], idxd[0], sem).wait()
            pltpu.async_copy(msg[0], acc_sh.at[idxd[0]], sem_s[0], add=True)

        pltpu.make_async_copy(msg[0], acc_sh.at[idxd[0]], sem_s[0]).wait()
        pltpu.make_async_copy(msg[1], acc_sh.at[idxd[1]], sem_s[1]).wait()

        if tail:
            tl = tloc + full_iters * CHUNK
            tb = tbase + full_iters * CHUNK
            pltpu.sync_copy(src_hbm.at[pl.ds(tb, tail)], idxs8)
            pltpu.sync_copy(dst_hbm.at[pl.ds(tb, tail)], idxd8)
            pltpu.sync_copy(w_hbm.at[pl.ds(tl, tail)], w0.at[pl.ds(0, tail)])
            pltpu.async_copy(
                x_hbm.at[idxs8], hsrc0.at[pl.ds(0, tail)], sem).wait()
            lax.fori_loop(0, tail, make_edge_body(hsrc0, w0, msg0), 0)
            pltpu.sync_copy(msg0.at[pl.ds(0, tail)], acc_sh.at[idxd8], add=True)

        plsc.subcore_barrier()

        pltpu.sync_copy(
            acc_sh.at[pl.ds(sid * ROWS_PER_TILE, ROWS_PER_TILE)],
            out_hbm.at[cid, pl.ds(sid * ROWS_PER_TILE, ROWS_PER_TILE)])

    return body(x, src, dst, wext)


def _combine_body(pa_ref, pb_ref, o_ref):
    a = pa_ref[0] + pa_ref[1] + pb_ref[0] + pb_ref[1]
    deg = jnp.maximum(a[:, WEXT:WEXT + 1], 1.0)
    s = a[:, :NS] / deg
    v = a[:, NS:WEXT] / deg
    r = lax.broadcasted_iota(jnp.int32, (3 * NV, 3 * NV), 0)
    c = lax.broadcasted_iota(jnp.int32, (3 * NV, 3 * NV), 1)
    perm = ((c % 3) * NV + (c // 3) == r).astype(jnp.float32)
    vp = jnp.dot(v, perm, preferred_element_type=jnp.float32)
    o_ref[...] = jnp.concatenate([s, vp], axis=1)


def _combine(pa, pb):
    grid = (10,)
    blk = N_NODES // 10
    spec = pl.BlockSpec((2, blk, MROW), lambda i: (0, i, 0))
    return pl.pallas_call(
        _combine_body,
        grid=grid,
        in_specs=[spec, spec],
        out_specs=pl.BlockSpec((blk, NS + 3 * NV), lambda i: (i, 0)),
        out_shape=jax.ShapeDtypeStruct((N_NODES, NS + 3 * NV), jnp.float32),
    )(pa, pb)


def kernel(x, edge_index, edge_attr, edge_sh, W1, b1, W2, b2):
    src = edge_index[0].astype(jnp.int32)
    dst = edge_index[1].astype(jnp.int32)
    ea_t = edge_attr.T
    sh_t = edge_sh.T
    b1r = b1.reshape(1, HID)
    b2r = b2.reshape(1, NS + NV)
    x_pad = jnp.pad(x, ((0, 0), (0, XW - NS)))

    wext_a = _mlp(ea_t, sh_t, W1, b1r, W2, b2r, SPLIT, 0)
    pa = _sc_kernel(x_pad, src, dst, wext_a, 0, SPLIT)
    wext_b = _mlp(ea_t, sh_t, W1, b1r, W2, b2r, N_EDGES - SPLIT,
                  SPLIT // MLP_BLK)
    pb = _sc_kernel(x_pad, src, dst, wext_b, SPLIT, N_EDGES - SPLIT)
    return _combine(pa, pb)


# speedup vs baseline: 1.5476x; 1.1668x over previous
import functools

import jax
import jax.numpy as jnp
from jax import lax
from jax.experimental import pallas as pl
from jax.experimental.pallas import tpu as pltpu
from jax.experimental.pallas import tpu_sc as plsc

NS = 64
NV = 16
HID = 192
N_NODES = 10000
N_EDGES = 160000

WEXT = NS + 3 * NV
MROW = 128
NTILES = 32
CHUNK = 64
ACC_ROWS = 10112
ROWS_PER_TILE = ACC_ROWS // 16
MLP_BLK = 3200
XW = 128
SPLIT = 89600


def _mlp_body(at_ref, sht_ref, w1_ref, b1_ref, w2_ref, b2_ref, o_ref):
    ht = jnp.maximum(
        lax.dot_general(w1_ref[...], at_ref[...], (((0,), (0,)), ((), ())),
                        preferred_element_type=jnp.float32) + b1_ref[...], 0.0)
    wt = lax.dot_general(w2_ref[...], ht, (((0,), (0,)), ((), ())),
                         preferred_element_type=jnp.float32) + b2_ref[...]
    wst = wt[:NS, :]
    wvt = wt[NS:NS + NV, :]
    u0 = wvt * sht_ref[1:2, :]
    u1 = wvt * sht_ref[2:3, :]
    u2 = wvt * sht_ref[3:4, :]
    pad = jnp.zeros((MROW - WEXT, MLP_BLK), jnp.float32)
    ot = jnp.concatenate([wst, u0, u1, u2, pad], axis=0)
    o_ref[...] = ot.T


def _mlp(edge_attr_t, edge_sh_t, W1, b1, W2, b2, n_edges, blk_off):
    grid = (n_edges // MLP_BLK,)
    return pl.pallas_call(
        _mlp_body,
        grid=grid,
        in_specs=[
            pl.BlockSpec((HID, MLP_BLK), lambda i: (0, i + blk_off)),
            pl.BlockSpec((4, MLP_BLK), lambda i: (0, i + blk_off)),
            pl.BlockSpec((HID, HID), lambda i: (0, 0)),
            pl.BlockSpec((HID, 1), lambda i: (0, 0)),
            pl.BlockSpec((HID, NS + NV), lambda i: (0, 0)),
            pl.BlockSpec((NS + NV, 1), lambda i: (0, 0)),
        ],
        out_specs=pl.BlockSpec((MLP_BLK, MROW), lambda i: (i, 0)),
        out_shape=jax.ShapeDtypeStruct((n_edges, MROW), jnp.float32),
    )(edge_attr_t, edge_sh_t, W1, b1, W2, b2)


def _sc_kernel(x, src, dst, wext, ebase, n_edges):
    ept = n_edges // NTILES
    full_iters = ept // CHUNK
    pairs = full_iters // 2
    epi = full_iters - 2 * pairs
    tail = ept - full_iters * CHUNK
    assert ept % 8 == 0 and (tail == 0 or tail >= 8) and pairs >= 2
    mesh = plsc.VectorSubcoreMesh(core_axis_name="c", subcore_axis_name="s")

    @functools.partial(
        pl.kernel,
        mesh=mesh,
        out_type=jax.ShapeDtypeStruct((2, ACC_ROWS, MROW), jnp.float32),
        scratch_types=[
            pltpu.VMEM((CHUNK,), jnp.int32),
            pltpu.VMEM((CHUNK,), jnp.int32),
            pltpu.VMEM((CHUNK,), jnp.int32),
            pltpu.VMEM((CHUNK,), jnp.int32),
            pltpu.VMEM((max(tail, 8),), jnp.int32),
            pltpu.VMEM((max(tail, 8),), jnp.int32),
            pltpu.VMEM((CHUNK, XW), jnp.float32),
            pltpu.VMEM((CHUNK, XW), jnp.float32),
            pltpu.VMEM((CHUNK, MROW), jnp.float32),
            pltpu.VMEM((CHUNK, MROW), jnp.float32),
            pltpu.VMEM((CHUNK, MROW), jnp.float32),
            pltpu.VMEM((CHUNK, MROW), jnp.float32),
            pltpu.VMEM_SHARED((ACC_ROWS, MROW), jnp.float32),
            pltpu.SemaphoreType.DMA,
            pltpu.SemaphoreType.DMA,
            pltpu.SemaphoreType.DMA,
            pltpu.SemaphoreType.DMA,
            pltpu.SemaphoreType.DMA,
            pltpu.SemaphoreType.DMA,
            pltpu.SemaphoreType.DMA,
            pltpu.SemaphoreType.DMA,
            pltpu.SemaphoreType.DMA,
        ],
    )
    def body(x_hbm, src_hbm, dst_hbm, w_hbm, out_hbm,
             idxs0, idxs1, idxd0, idxd1, idxs8, idxd8,
             hsrc0, hsrc1, w0, w1, msg0, msg1, acc_sh,
             sem_in0, sem_in1, sem_w0, sem_w1, sem_g0, sem_g1,
             sem_s0, sem_s1, sem):
        cid = lax.axis_index("c")
        sid = lax.axis_index("s")
        idxs = (idxs0, idxs1)
        idxd = (idxd0, idxd1)
        hsrc = (hsrc0, hsrc1)
        wv = (w0, w1)
        msg = (msg0, msg1)
        sem_in = (sem_in0, sem_in1)
        sem_w = (sem_w0, sem_w1)
        sem_g = (sem_g0, sem_g1)
        sem_s = (sem_s0, sem_s1)

        zeros16 = jnp.zeros((16,), jnp.float32)

        def zero_buf(buf):
            def f(i, _):
                r = i // (MROW // 16)
                c = i % (MROW // 16)
                buf[r, pl.ds(c * 16, 16)] = zeros16
                return 0
            lax.fori_loop(0, CHUNK * (MROW // 16), f, 0)

        zero_buf(msg0)
        zero_buf(msg1)

        for k in range(ROWS_PER_TILE // CHUNK):
            pltpu.sync_copy(
                msg0, acc_sh.at[pl.ds(sid * ROWS_PER_TILE + k * CHUNK, CHUNK)])
        rem = ROWS_PER_TILE % CHUNK
        if rem:
            pltpu.sync_copy(
                msg0.at[pl.ds(0, rem)],
                acc_sh.at[pl.ds(sid * ROWS_PER_TILE
                                + (ROWS_PER_TILE // CHUNK) * CHUNK, rem)])

        ii = lax.broadcasted_iota(jnp.int32, (16,), 0)
        cvec = jnp.where(ii == 0, 1.0, 0.0).astype(jnp.float32)

        def crow(buf):
            def f(r, _):
                buf[r, pl.ds(WEXT, 16)] = cvec
                return 0
            lax.fori_loop(0, CHUNK, f, 0)

        crow(msg0)
        crow(msg1)

        plsc.subcore_barrier()

        tloc = (cid * 16 + sid) * ept
        tbase = ebase + tloc

        def make_edge_body(hs, wb, mb):
            def edge_body(e, _):
                h0 = hs[e, pl.ds(0, 16)]
                mb[e, pl.ds(0, 16)] = h0 * wb[e, pl.ds(0, 16)]
                for k in range(1, NS // 16):
                    mb[e, pl.ds(k * 16, 16)] = (
                        hs[e, pl.ds(k * 16, 16)] * wb[e, pl.ds(k * 16, 16)])
                for j in range(3):
                    off = NS + j * 16
                    mb[e, pl.ds(off, 16)] = h0 * wb[e, pl.ds(off, 16)]
                return 0
            return edge_body

        def half_step(p, c, S, prefetch):
            pltpu.make_async_copy(x_hbm.at[idxs[S]], hsrc[S], sem_g[S]).wait()
            pltpu.make_async_copy(
                w_hbm.at[pl.ds(0, CHUNK)], wv[S], sem_w[S]).wait()

            @pl.when(p > 0)
            def _():
                pltpu.make_async_copy(
                    msg[S], acc_sh.at[idxd[S]], sem_s[S]).wait()

            base = tbase + c * CHUNK
            pltpu.async_copy(dst_hbm.at[pl.ds(base, CHUNK)], idxd[S], sem)

            @pl.when(prefetch)
            def _():
                nbase = tbase + (c + 2) * CHUNK
                pltpu.async_copy(
                    src_hbm.at[pl.ds(nbase, CHUNK)], idxs[S], sem_in[S])

            lax.fori_loop(0, CHUNK, make_edge_body(hsrc[S], wv[S], msg[S]), 0)

            pltpu.make_async_copy(
                dst_hbm.at[pl.ds(base, CHUNK)], idxd[S], sem).wait()
            pltpu.async_copy(msg[S], acc_sh.at[idxd[S]], sem_s[S], add=True)

            @pl.when(prefetch)
            def _():
                nloc = tloc + (c + 2) * CHUNK
                nbase = tbase + (c + 2) * CHUNK
                pltpu.async_copy(
                    w_hbm.at[pl.ds(nloc, CHUNK)], wv[S], sem_w[S])
                pltpu.make_async_copy(
                    src_hbm.at[pl.ds(nbase, CHUNK)], idxs[S], sem_in[S]).wait()
                pltpu.async_copy(x_hbm.at[idxs[S]], hsrc[S], sem_g[S])

        def pair_body(p, _):
            half_step(p, 2 * p, 0, p < pairs - 1 + epi)
            half_step(p, 2 * p + 1, 1, p < pairs - 1)
            return 0

        for S in range(2):
            pltpu.async_copy(
                src_hbm.at[pl.ds(tbase + S * CHUNK, CHUNK)], idxs[S], sem_in[S])
            pltpu.async_copy(
                w_hbm.at[pl.ds(tloc + S * CHUNK, CHUNK)], wv[S], sem_w[S])
        for S in range(2):
            pltpu.make_async_copy(
                src_hbm.at[pl.ds(tbase, CHUNK)], idxs[S], sem_in[S]).wait()
            pltpu.async_copy(x_hbm.at[idxs[S]], hsrc[S], sem_g[S])

        lax.fori_loop(0, pairs, pair_body, 0)

        if epi:
            ce = 2 * pairs
            pltpu.make_async_copy(x_hbm.at[idxs[0]], hsrc[0], sem_g[0]).wait()
            pltpu.make_async_copy(
                w_hbm.at[pl.ds(0, CHUNK)], wv[0], sem_w[0]).wait()
            pltpu.make_async_copy(msg[0], acc_sh.at[idxd[0]], sem_s[0]).wait()
            ebch = tbase + ce * CHUNK
            pltpu.async_copy(dst_hbm.at[pl.ds(ebch, CHUNK)], idxd[0], sem)
            lax.fori_loop(0, CHUNK, make_edge_body(hsrc[0], wv[0], msg[0]), 0)
            pltpu.make_async_copy(
                dst_hbm.at[pl.ds(ebch, CHUNK)], idxd[0], sem).wait()
            pltpu.async_copy(msg[0], acc_sh.at[idxd[0]], sem_s[0], add=True)

        pltpu.make_async_copy(msg[0], acc_sh.at[idxd[0]], sem_s[0]).wait()
        pltpu.make_async_copy(msg[1], acc_sh.at[idxd[1]], sem_s[1]).wait()

        if tail:
            tl = tloc + full_iters * CHUNK
            tb = tbase + full_iters * CHUNK
            pltpu.sync_copy(src_hbm.at[pl.ds(tb, tail)], idxs8)
            pltpu.sync_copy(dst_hbm.at[pl.ds(tb, tail)], idxd8)
            pltpu.sync_copy(w_hbm.at[pl.ds(tl, tail)], w0.at[pl.ds(0, tail)])
            pltpu.async_copy(
                x_hbm.at[idxs8], hsrc0.at[pl.ds(0, tail)], sem).wait()
            lax.fori_loop(0, tail, make_edge_body(hsrc0, w0, msg0), 0)
            pltpu.sync_copy(msg0.at[pl.ds(0, tail)], acc_sh.at[idxd8], add=True)

        plsc.subcore_barrier()

        pltpu.sync_copy(
            acc_sh.at[pl.ds(sid * ROWS_PER_TILE, ROWS_PER_TILE)],
            out_hbm.at[cid, pl.ds(sid * ROWS_PER_TILE, ROWS_PER_TILE)])

    return body(x, src, dst, wext)


def _combine_body(pa_ref, pb_ref, o_ref):
    a = pa_ref[0] + pa_ref[1] + pb_ref[0] + pb_ref[1]
    deg = jnp.maximum(a[:, WEXT:WEXT + 1], 1.0)
    s = a[:, :NS] / deg
    v = a[:, NS:WEXT] / deg
    r = lax.broadcasted_iota(jnp.int32, (3 * NV, 3 * NV), 0)
    c = lax.broadcasted_iota(jnp.int32, (3 * NV, 3 * NV), 1)
    perm = ((c % 3) * NV + (c // 3) == r).astype(jnp.float32)
    vp = jnp.dot(v, perm, preferred_element_type=jnp.float32)
    o_ref[...] = jnp.concatenate([s, vp], axis=1)


def _combine(pa, pb):
    grid = (10,)
    blk = N_NODES // 10
    spec = pl.BlockSpec((2, blk, MROW), lambda i: (0, i, 0))
    return pl.pallas_call(
        _combine_body,
        grid=grid,
        in_specs=[spec, spec],
        out_specs=pl.BlockSpec((blk, NS + 3 * NV), lambda i: (i, 0)),
        out_shape=jax.ShapeDtypeStruct((N_NODES, NS + 3 * NV), jnp.float32),
    )(pa, pb)


def kernel(x, edge_index, edge_attr, edge_sh, W1, b1, W2, b2):
    src = edge_index[0].astype(jnp.int32)
    dst = edge_index[1].astype(jnp.int32)
    ea_t = edge_attr.T
    sh_t = edge_sh.T
    b1r = b1.reshape(HID, 1)
    b2r = b2.reshape(NS + NV, 1)
    x_pad = jnp.pad(x, ((0, 0), (0, XW - NS)))

    wext_a = _mlp(ea_t, sh_t, W1, b1r, W2, b2r, SPLIT, 0)
    pa = _sc_kernel(x_pad, src, dst, wext_a, 0, SPLIT)
    wext_b = _mlp(ea_t, sh_t, W1, b1r, W2, b2r, N_EDGES - SPLIT,
                  SPLIT // MLP_BLK)
    pb = _sc_kernel(x_pad, src, dst, wext_b, SPLIT, N_EDGES - SPLIT)
    return _combine(pa, pb)


# speedup vs baseline: 1.5617x; 1.0091x over previous
import functools

import jax
import jax.numpy as jnp
from jax import lax
from jax.experimental import pallas as pl
from jax.experimental.pallas import tpu as pltpu
from jax.experimental.pallas import tpu_sc as plsc

NS = 64
NV = 16
HID = 192
N_NODES = 10000
N_EDGES = 160000

WEXT = NS + 3 * NV
MROW = 128
NTILES = 32
CHUNK = 64
ACC_ROWS = 10112
ROWS_PER_TILE = ACC_ROWS // 16
MLP_BLK = 3200
XW = 128
SPLIT = 76800


def _mlp_body(at_ref, sht_ref, w1_ref, b1_ref, w2_ref, b2_ref, o_ref):
    ht = jnp.maximum(
        lax.dot_general(w1_ref[...], at_ref[...], (((0,), (0,)), ((), ())),
                        preferred_element_type=jnp.float32) + b1_ref[...], 0.0)
    wt = lax.dot_general(w2_ref[...], ht, (((0,), (0,)), ((), ())),
                         preferred_element_type=jnp.float32) + b2_ref[...]
    wst = wt[:NS, :]
    wvt = wt[NS:NS + NV, :]
    u0 = wvt * sht_ref[1:2, :]
    u1 = wvt * sht_ref[2:3, :]
    u2 = wvt * sht_ref[3:4, :]
    pad = jnp.zeros((MROW - WEXT, MLP_BLK), jnp.float32)
    ot = jnp.concatenate([wst, u0, u1, u2, pad], axis=0)
    o_ref[...] = ot.T


def _mlp(edge_attr_t, edge_sh_t, W1, b1, W2, b2, n_edges, blk_off):
    grid = (n_edges // MLP_BLK,)
    return pl.pallas_call(
        _mlp_body,
        grid=grid,
        in_specs=[
            pl.BlockSpec((HID, MLP_BLK), lambda i: (0, i + blk_off)),
            pl.BlockSpec((4, MLP_BLK), lambda i: (0, i + blk_off)),
            pl.BlockSpec((HID, HID), lambda i: (0, 0)),
            pl.BlockSpec((HID, 1), lambda i: (0, 0)),
            pl.BlockSpec((HID, NS + NV), lambda i: (0, 0)),
            pl.BlockSpec((NS + NV, 1), lambda i: (0, 0)),
        ],
        out_specs=pl.BlockSpec((MLP_BLK, MROW), lambda i: (i, 0)),
        out_shape=jax.ShapeDtypeStruct((n_edges, MROW), jnp.float32),
    )(edge_attr_t, edge_sh_t, W1, b1, W2, b2)


def _sc_kernel(x, src, dst, wext, ebase, n_edges):
    ept = n_edges // NTILES
    full_iters = ept // CHUNK
    pairs = full_iters // 2
    epi = full_iters - 2 * pairs
    tail = ept - full_iters * CHUNK
    assert ept % 8 == 0 and (tail == 0 or tail >= 8) and pairs >= 2
    mesh = plsc.VectorSubcoreMesh(core_axis_name="c", subcore_axis_name="s")

    @functools.partial(
        pl.kernel,
        mesh=mesh,
        out_type=jax.ShapeDtypeStruct((2, ACC_ROWS, MROW), jnp.float32),
        scratch_types=[
            pltpu.VMEM((CHUNK,), jnp.int32),
            pltpu.VMEM((CHUNK,), jnp.int32),
            pltpu.VMEM((CHUNK,), jnp.int32),
            pltpu.VMEM((CHUNK,), jnp.int32),
            pltpu.VMEM((max(tail, 8),), jnp.int32),
            pltpu.VMEM((max(tail, 8),), jnp.int32),
            pltpu.VMEM((CHUNK, XW), jnp.float32),
            pltpu.VMEM((CHUNK, XW), jnp.float32),
            pltpu.VMEM((CHUNK, MROW), jnp.float32),
            pltpu.VMEM((CHUNK, MROW), jnp.float32),
            pltpu.VMEM((CHUNK, MROW), jnp.float32),
            pltpu.VMEM((CHUNK, MROW), jnp.float32),
            pltpu.VMEM_SHARED((ACC_ROWS, MROW), jnp.float32),
            pltpu.SemaphoreType.DMA,
            pltpu.SemaphoreType.DMA,
            pltpu.SemaphoreType.DMA,
            pltpu.SemaphoreType.DMA,
            pltpu.SemaphoreType.DMA,
            pltpu.SemaphoreType.DMA,
            pltpu.SemaphoreType.DMA,
            pltpu.SemaphoreType.DMA,
            pltpu.SemaphoreType.DMA,
        ],
    )
    def body(x_hbm, src_hbm, dst_hbm, w_hbm, out_hbm,
             idxs0, idxs1, idxd0, idxd1, idxs8, idxd8,
             hsrc0, hsrc1, w0, w1, msg0, msg1, acc_sh,
             sem_in0, sem_in1, sem_w0, sem_w1, sem_g0, sem_g1,
             sem_s0, sem_s1, sem):
        cid = lax.axis_index("c")
        sid = lax.axis_index("s")
        idxs = (idxs0, idxs1)
        idxd = (idxd0, idxd1)
        hsrc = (hsrc0, hsrc1)
        wv = (w0, w1)
        msg = (msg0, msg1)
        sem_in = (sem_in0, sem_in1)
        sem_w = (sem_w0, sem_w1)
        sem_g = (sem_g0, sem_g1)
        sem_s = (sem_s0, sem_s1)

        zeros16 = jnp.zeros((16,), jnp.float32)

        def zero_buf(buf):
            def f(i, _):
                r = i // (MROW // 16)
                c = i % (MROW // 16)
                buf[r, pl.ds(c * 16, 16)] = zeros16
                return 0
            lax.fori_loop(0, CHUNK * (MROW // 16), f, 0)

        zero_buf(msg0)
        zero_buf(msg1)

        for k in range(ROWS_PER_TILE // CHUNK):
            pltpu.sync_copy(
                msg0, acc_sh.at[pl.ds(sid * ROWS_PER_TILE + k * CHUNK, CHUNK)])
        rem = ROWS_PER_TILE % CHUNK
        if rem:
            pltpu.sync_copy(
                msg0.at[pl.ds(0, rem)],
                acc_sh.at[pl.ds(sid * ROWS_PER_TILE
                                + (ROWS_PER_TILE // CHUNK) * CHUNK, rem)])

        ii = lax.broadcasted_iota(jnp.int32, (16,), 0)
        cvec = jnp.where(ii == 0, 1.0, 0.0).astype(jnp.float32)

        def crow(buf):
            def f(r, _):
                buf[r, pl.ds(WEXT, 16)] = cvec
                return 0
            lax.fori_loop(0, CHUNK, f, 0)

        crow(msg0)
        crow(msg1)

        plsc.subcore_barrier()

        tloc = (cid * 16 + sid) * ept
        tbase = ebase + tloc

        def make_edge_body(hs, wb, mb):
            def edge_body(e, _):
                h0 = hs[e, pl.ds(0, 16)]
                mb[e, pl.ds(0, 16)] = h0 * wb[e, pl.ds(0, 16)]
                for k in range(1, NS // 16):
                    mb[e, pl.ds(k * 16, 16)] = (
                        hs[e, pl.ds(k * 16, 16)] * wb[e, pl.ds(k * 16, 16)])
                for j in range(3):
                    off = NS + j * 16
                    mb[e, pl.ds(off, 16)] = h0 * wb[e, pl.ds(off, 16)]
                return 0
            return edge_body

        def half_step(p, c, S, prefetch):
            pltpu.make_async_copy(x_hbm.at[idxs[S]], hsrc[S], sem_g[S]).wait()
            pltpu.make_async_copy(
                w_hbm.at[pl.ds(0, CHUNK)], wv[S], sem_w[S]).wait()

            @pl.when(p > 0)
            def _():
                pltpu.make_async_copy(
                    msg[S], acc_sh.at[idxd[S]], sem_s[S]).wait()

            base = tbase + c * CHUNK
            pltpu.async_copy(dst_hbm.at[pl.ds(base, CHUNK)], idxd[S], sem)

            @pl.when(prefetch)
            def _():
                nbase = tbase + (c + 2) * CHUNK
                pltpu.async_copy(
                    src_hbm.at[pl.ds(nbase, CHUNK)], idxs[S], sem_in[S])

            lax.fori_loop(0, CHUNK, make_edge_body(hsrc[S], wv[S], msg[S]), 0)

            pltpu.make_async_copy(
                dst_hbm.at[pl.ds(base, CHUNK)], idxd[S], sem).wait()
            pltpu.async_copy(msg[S], acc_sh.at[idxd[S]], sem_s[S], add=True)

            @pl.when(prefetch)
            def _():
                nloc = tloc + (c + 2) * CHUNK
                nbase = tbase + (c + 2) * CHUNK
                pltpu.async_copy(
                    w_hbm.at[pl.ds(nloc, CHUNK)], wv[S], sem_w[S])
                pltpu.make_async_copy(
                    src_hbm.at[pl.ds(nbase, CHUNK)], idxs[S], sem_in[S]).wait()
                pltpu.async_copy(x_hbm.at[idxs[S]], hsrc[S], sem_g[S])

        def pair_body(p, _):
            half_step(p, 2 * p, 0, p < pairs - 1 + epi)
            half_step(p, 2 * p + 1, 1, p < pairs - 1)
            return 0

        for S in range(2):
            pltpu.async_copy(
                src_hbm.at[pl.ds(tbase + S * CHUNK, CHUNK)], idxs[S], sem_in[S])
            pltpu.async_copy(
                w_hbm.at[pl.ds(tloc + S * CHUNK, CHUNK)], wv[S], sem_w[S])
        for S in range(2):
            pltpu.make_async_copy(
                src_hbm.at[pl.ds(tbase, CHUNK)], idxs[S], sem_in[S]).wait()
            pltpu.async_copy(x_hbm.at[idxs[S]], hsrc[S], sem_g[S])

        lax.fori_loop(0, pairs, pair_body, 0)

        if epi:
            ce = 2 * pairs
            pltpu.make_async_copy(x_hbm.at[idxs[0]], hsrc[0], sem_g[0]).wait()
            pltpu.make_async_copy(
                w_hbm.at[pl.ds(0, CHUNK)], wv[0], sem_w[0]).wait()
            pltpu.make_async_copy(msg[0], acc_sh.at[idxd[0]], sem_s[0]).wait()
            ebch = tbase + ce * CHUNK
            pltpu.async_copy(dst_hbm.at[pl.ds(ebch, CHUNK)], idxd[0], sem)
            lax.fori_loop(0, CHUNK, make_edge_body(hsrc[0], wv[0], msg[0]), 0)
            pltpu.make_async_copy(
                dst_hbm.at[pl.ds(ebch, CHUNK)], idxd[0], sem).wait()
            pltpu.async_copy(msg[0], acc_sh.at[idxd[0]], sem_s[0], add=True)

        pltpu.make_async_copy(msg[0], acc_sh.at[idxd[0]], sem_s[0]).wait()
        pltpu.make_async_copy(msg[1], acc_sh.at[idxd[1]], sem_s[1]).wait()

        if tail:
            tl = tloc + full_iters * CHUNK
            tb = tbase + full_iters * CHUNK
            pltpu.sync_copy(src_hbm.at[pl.ds(tb, tail)], idxs8)
            pltpu.sync_copy(dst_hbm.at[pl.ds(tb, tail)], idxd8)
            pltpu.sync_copy(w_hbm.at[pl.ds(tl, tail)], w0.at[pl.ds(0, tail)])
            pltpu.async_copy(
                x_hbm.at[idxs8], hsrc0.at[pl.ds(0, tail)], sem).wait()
            lax.fori_loop(0, tail, make_edge_body(hsrc0, w0, msg0), 0)
            pltpu.sync_copy(msg0.at[pl.ds(0, tail)], acc_sh.at[idxd8], add=True)

        plsc.subcore_barrier()

        pltpu.sync_copy(
            acc_sh.at[pl.ds(sid * ROWS_PER_TILE, ROWS_PER_TILE)],
            out_hbm.at[cid, pl.ds(sid * ROWS_PER_TILE, ROWS_PER_TILE)])

    return body(x, src, dst, wext)


def _combine_body(pa_ref, pb_ref, o_ref):
    a = pa_ref[0] + pa_ref[1] + pb_ref[0] + pb_ref[1]
    deg = jnp.maximum(a[:, WEXT:WEXT + 1], 1.0)
    s = a[:, :NS] / deg
    v = a[:, NS:WEXT] / deg
    r = lax.broadcasted_iota(jnp.int32, (3 * NV, 3 * NV), 0)
    c = lax.broadcasted_iota(jnp.int32, (3 * NV, 3 * NV), 1)
    perm = ((c % 3) * NV + (c // 3) == r).astype(jnp.float32)
    vp = jnp.dot(v, perm, preferred_element_type=jnp.float32)
    o_ref[...] = jnp.concatenate([s, vp], axis=1)


def _combine(pa, pb):
    grid = (10,)
    blk = N_NODES // 10
    spec = pl.BlockSpec((2, blk, MROW), lambda i: (0, i, 0))
    return pl.pallas_call(
        _combine_body,
        grid=grid,
        in_specs=[spec, spec],
        out_specs=pl.BlockSpec((blk, NS + 3 * NV), lambda i: (i, 0)),
        out_shape=jax.ShapeDtypeStruct((N_NODES, NS + 3 * NV), jnp.float32),
    )(pa, pb)


def kernel(x, edge_index, edge_attr, edge_sh, W1, b1, W2, b2):
    src = edge_index[0].astype(jnp.int32)
    dst = edge_index[1].astype(jnp.int32)
    ea_t = edge_attr.T
    sh_t = edge_sh.T
    b1r = b1.reshape(HID, 1)
    b2r = b2.reshape(NS + NV, 1)
    x_pad = jnp.pad(x, ((0, 0), (0, XW - NS)))

    wext_a = _mlp(ea_t, sh_t, W1, b1r, W2, b2r, SPLIT, 0)
    pa = _sc_kernel(x_pad, src, dst, wext_a, 0, SPLIT)
    wext_b = _mlp(ea_t, sh_t, W1, b1r, W2, b2r, N_EDGES - SPLIT,
                  SPLIT // MLP_BLK)
    pb = _sc_kernel(x_pad, src, dst, wext_b, SPLIT, N_EDGES - SPLIT)
    return _combine(pa, pb)
